# baseline (speedup 1.0000x reference)
import jax
import jax.numpy as jnp
from jax import lax
from jax.experimental import pallas as pl
from jax.experimental.pallas import tpu as pltpu
from jax.experimental.pallas import tpu_sc as plsc

VOCAB = 1000000
EMBED_DIM = 64
BATCH = 16384
FIELDS = 26

NC = 2
NS = 16
NW = NC * NS

NBT = BATCH // 128
NTC = FIELDS * NBT
PER_W = NTC // NW
LANE = 128


def _splat(v):
    return jnp.full((16,), v, jnp.int32)


def _transpose_tile(rows_v, cols_v, iota16):
    lanes = [iota16 + (lg * 16) for lg in range(8)]
    for d in range(EMBED_DIM):
        cd = _splat(d)
        for lg in range(8):
            vec = plsc.load_gather(rows_v, [lanes[lg], cd])
            cols_v[d >> 3, d & 7, pl.ds(lg * 16, 16)] = vec


def _body(weight_hbm, idx_hbm, out_hbm,
          idx_v, rows0, rows1, cols0, cols1, sg0, sg1, so0, so1):
    wid = lax.axis_index("s") * NC + lax.axis_index("c")
    pltpu.sync_copy(idx_hbm.at[wid], idx_v)
    iota16 = lax.broadcasted_iota(jnp.int32, (16,), 0)

    rows = (rows0, rows1)
    cols = (cols0, cols1)
    sg = (sg0, sg1)
    so = (so0, so1)

    pltpu.async_copy(weight_hbm.at[idx_v.at[0]], rows0, sg0)
    pltpu.async_copy(weight_hbm.at[idx_v.at[1]], rows1, sg1)

    def pair(it, carry):
        for p in range(2):
            j = 2 * it + p
            t = wid * PER_W + j
            f = t // NBT
            bt = t - f * NBT
            out_slice = out_hbm.at[f, :, bt]
            pltpu.make_async_copy(
                weight_hbm.at[idx_v.at[j]], rows[p], sg[p]).wait()
            @pl.when(j >= 2)
            def _():
                pltpu.make_async_copy(cols[p], out_slice, so[p]).wait()
            _transpose_tile(rows[p], cols[p], iota16)
            @pl.when(j + 2 < PER_W)
            def _():
                pltpu.async_copy(
                    weight_hbm.at[idx_v.at[j + 2]], rows[p], sg[p])
            pltpu.async_copy(cols[p], out_slice, so[p])
        return carry

    lax.fori_loop(0, PER_W // 2, pair, 0)

    for p in range(2):
        j = PER_W - 2 + p
        t = wid * PER_W + j
        f = t // NBT
        bt = t - f * NBT
        pltpu.make_async_copy(cols[p], out_hbm.at[f, :, bt], so[p]).wait()


@jax.jit
def _embed(idx, weight):
    mesh = plsc.VectorSubcoreMesh(core_axis_name="c", subcore_axis_name="s")
    k = pl.kernel(
        _body,
        out_type=jax.ShapeDtypeStruct((FIELDS, 8, NBT, 8, LANE), jnp.float32),
        mesh=mesh,
        scratch_types=[
            pltpu.VMEM((PER_W, LANE), jnp.int32),
            pltpu.VMEM((LANE, EMBED_DIM), jnp.float32),
            pltpu.VMEM((LANE, EMBED_DIM), jnp.float32),
            pltpu.VMEM((8, 8, LANE), jnp.float32),
            pltpu.VMEM((8, 8, LANE), jnp.float32),
            pltpu.SemaphoreType.DMA,
            pltpu.SemaphoreType.DMA,
            pltpu.SemaphoreType.DMA,
            pltpu.SemaphoreType.DMA,
        ],
        compiler_params=pltpu.CompilerParams(
            use_tc_tiling_on_sc=False, needs_layout_passes=False),
    )
    return k(weight, idx)


def kernel(input, weight):
    idx = input.astype(jnp.int32).T.reshape(NW, PER_W, LANE)
    out5 = _embed(idx, weight)
    return out5.transpose(2, 4, 0, 1, 3).reshape(BATCH, FIELDS, EMBED_DIM)


# speedup vs baseline: 1.2578x; 1.2578x over previous
import jax
import jax.numpy as jnp
from jax import lax
from jax.experimental import pallas as pl
from jax.experimental.pallas import tpu as pltpu
from jax.experimental.pallas import tpu_sc as plsc

VOCAB = 1000000
EMBED_DIM = 64
BATCH = 16384
FIELDS = 26

NC = 2
NS = 16
NW = NC * NS

NBT = BATCH // 128
NTC = FIELDS * NBT
PER_W = NTC // NW
LANE = 128


def _transpose_tile(rows_v, cols_v, pregs):
    def rstep(l4, carry):
        for r in range(4):
            l = l4 * 4 + r
            for g in range(4):
                vec = rows_v[l, pl.ds(g * 16, 16)]
                plsc.store_scatter(cols_v, [pregs[g] + l], vec)
        return carry

    lax.fori_loop(0, 32, rstep, 0)


def _body(weight_hbm, idx_hbm, out_hbm,
          idx_v, rows0, rows1, cols0, cols1, sg0, sg1, so0, so1):
    wid = lax.axis_index("s") * NC + lax.axis_index("c")
    pltpu.sync_copy(idx_hbm.at[wid], idx_v)
    iota16 = lax.broadcasted_iota(jnp.int32, (16,), 0)
    pregs = [(iota16 + g * 16) * 128 for g in range(4)]

    rows = (rows0, rows1)
    cols = (cols0, cols1)
    sg = (sg0, sg1)
    so = (so0, so1)

    pltpu.async_copy(weight_hbm.at[idx_v.at[0]], rows0, sg0)
    pltpu.async_copy(weight_hbm.at[idx_v.at[1]], rows1, sg1)

    def pair(it, carry):
        for p in range(2):
            j = 2 * it + p
            t = wid * PER_W + j
            f = t // NBT
            bt = t - f * NBT
            pltpu.make_async_copy(
                weight_hbm.at[idx_v.at[j]], rows[p], sg[p]).wait()
            @pl.when(j >= 2)
            def _():
                for dt in range(8):
                    pltpu.make_async_copy(
                        cols[p].at[pl.ds(dt * 1024, 1024)],
                        out_hbm.at[f, dt, bt], so[p]).wait()
            _transpose_tile(rows[p], cols[p], pregs)
            @pl.when(j + 2 < PER_W)
            def _():
                pltpu.async_copy(
                    weight_hbm.at[idx_v.at[j + 2]], rows[p], sg[p])
            for dt in range(8):
                pltpu.async_copy(cols[p].at[pl.ds(dt * 1024, 1024)],
                                 out_hbm.at[f, dt, bt], so[p])
        return carry

    lax.fori_loop(0, PER_W // 2, pair, 0)

    for p in range(2):
        j = PER_W - 2 + p
        t = wid * PER_W + j
        f = t // NBT
        bt = t - f * NBT
        for dt in range(8):
            pltpu.make_async_copy(cols[p].at[pl.ds(dt * 1024, 1024)],
                                  out_hbm.at[f, dt, bt], so[p]).wait()


@jax.jit
def _embed(idx, weight_padded):
    mesh = plsc.VectorSubcoreMesh(core_axis_name="c", subcore_axis_name="s")
    k = pl.kernel(
        _body,
        out_type=jax.ShapeDtypeStruct((FIELDS, 8, NBT, 1024), jnp.float32),
        mesh=mesh,
        scratch_types=[
            pltpu.VMEM((PER_W, LANE), jnp.int32),
            pltpu.VMEM((LANE, LANE), jnp.float32),
            pltpu.VMEM((LANE, LANE), jnp.float32),
            pltpu.VMEM((8192,), jnp.float32),
            pltpu.VMEM((8192,), jnp.float32),
            pltpu.SemaphoreType.DMA,
            pltpu.SemaphoreType.DMA,
            pltpu.SemaphoreType.DMA,
            pltpu.SemaphoreType.DMA,
        ],
        compiler_params=pltpu.CompilerParams(
            use_tc_tiling_on_sc=False, needs_layout_passes=False),
    )
    return k(weight_padded, idx)


def kernel(input, weight):
    idx = input.astype(jnp.int32).T.reshape(NW, PER_W, LANE)
    wp = jnp.pad(weight, ((0, 0), (0, LANE - EMBED_DIM)))
    out5 = _embed(idx, wp)
    return (out5.reshape(FIELDS, 8, NBT, 8, LANE)
            .transpose(2, 4, 0, 1, 3)
            .reshape(BATCH, FIELDS, EMBED_DIM))


# speedup vs baseline: 1.4316x; 1.1382x over previous
import jax
import jax.numpy as jnp
from jax import lax
from jax.experimental import pallas as pl
from jax.experimental.pallas import tpu as pltpu
from jax.experimental.pallas import tpu_sc as plsc

VOCAB = 1000000
EMBED_DIM = 64
BATCH = 16384
FIELDS = 26

NC = 2
NS = 16
NW = NC * NS

TOTAL = BATCH * FIELDS
PER_W = TOTAL // NW
CHUNK = 128
NCHUNK = PER_W // CHUNK
LANE = 128
NBUF = 4
NGROUP = NCHUNK // NBUF


def _body(weight_hbm, idx_hbm, out_hbm, idx_v, *scr):
    rows = scr[:NBUF]
    sg = scr[NBUF:2 * NBUF]
    so = scr[2 * NBUF:]
    wid = lax.axis_index("s") * NC + lax.axis_index("c")
    base = wid * PER_W
    pltpu.sync_copy(idx_hbm.at[wid], idx_v)

    for c in range(NBUF):
        pltpu.async_copy(weight_hbm.at[idx_v.at[c]], rows[c], sg[c])

    def group(g, carry):
        j0 = g * NBUF
        for c in range(NBUF):
            j = j0 + c
            dst = out_hbm.at[pl.ds(base + j * CHUNK, CHUNK)]
            src = rows[c].at[:, pl.ds(0, EMBED_DIM)]
            pltpu.make_async_copy(
                weight_hbm.at[idx_v.at[j]], rows[c], sg[c]).wait()
            @pl.when(j >= NBUF)
            def _():
                pltpu.make_async_copy(src, dst, so[c]).wait()
            pltpu.async_copy(src, dst, so[c])
            @pl.when(j + NBUF < NCHUNK)
            def _():
                pltpu.async_copy(
                    weight_hbm.at[idx_v.at[j + NBUF]], rows[c], sg[c])
        return carry

    lax.fori_loop(0, NGROUP, group, 0)

    for c in range(NBUF):
        j = (NGROUP - 1) * NBUF + c
        pltpu.make_async_copy(
            rows[c].at[:, pl.ds(0, EMBED_DIM)],
            out_hbm.at[pl.ds(base + j * CHUNK, CHUNK)], so[c]).wait()


@jax.jit
def _embed(idx, weight_padded):
    mesh = plsc.VectorSubcoreMesh(core_axis_name="c", subcore_axis_name="s")
    k = pl.kernel(
        _body,
        out_type=jax.ShapeDtypeStruct((TOTAL, EMBED_DIM), jnp.float32),
        mesh=mesh,
        scratch_types=(
            [pltpu.VMEM((NCHUNK, CHUNK), jnp.int32)]
            + [pltpu.VMEM((CHUNK, LANE), jnp.float32) for _ in range(NBUF)]
            + [pltpu.SemaphoreType.DMA for _ in range(2 * NBUF)]
        ),
        compiler_params=pltpu.CompilerParams(
            use_tc_tiling_on_sc=False, needs_layout_passes=False),
    )
    return k(weight_padded, idx)


def kernel(input, weight):
    idx = input.astype(jnp.int32).reshape(NW, NCHUNK, CHUNK)
    wp = jnp.pad(weight, ((0, 0), (0, LANE - EMBED_DIM)))
    out = _embed(idx, wp)
    return out.reshape(BATCH, FIELDS, EMBED_DIM)
